# baseline (speedup 1.0000x reference)
import functools

import jax
import jax.numpy as jnp
from jax import lax
from jax.experimental import pallas as pl
from jax.experimental.pallas import tpu as pltpu
from jax.experimental.pallas import tpu_sc as plsc

N = 10000
E = 320000
F = 128
NC = 2
NS = 16
NW = NC * NS
KCH = 79
EPW = KCH * 128
E_PAD = NW * EPW
RA = E_PAD // 128
LB = 4096
N_ACC = 10240
TRASH = 10100
CB = 400


def _edge_tables(wmT, embT, bm2, wqT, bq2, wkT, bk2):
    f32 = jnp.float32
    e0c = embT[:, 0:1]
    e1c = embT[:, 1:2]
    dcc = e1c - e0c
    WrT = wmT[:, 0:5]
    WsT = wmT[:, 5:10]
    wc_col = wmT[:, 10:11]
    dot = functools.partial(jnp.dot, preferred_element_type=f32)
    Acol = dot(WrT, e0c) + dot(WsT, e0c) + bm2
    Brc = dot(WrT, dcc)
    Bsc = dot(WsT, dcc)
    q0 = dot(wqT, e0c) + bq2
    dq = dot(wqT, dcc)
    k0 = dot(wkT, e0c) + bk2
    dk = dot(wkT, dcc)
    base = [Acol, Acol + Bsc, Acol + Brc, Acol + Brc + Bsc]
    sig = []
    for ir in (0, 1):
        for is_ in (0, 1):
            a = jnp.sum((q0 + is_ * dq) * (k0 + ir * dk))
            sig.append(1.0 / (1.0 + jnp.exp(-a)))
    return base, wc_col, sig


def _edge_values_body(c_ref, wmT_ref, embT_ref, bm2_ref, wqT_ref, bq2_ref,
                      wkT_ref, bk2_ref, v_ref):
    bf16 = jnp.bfloat16
    base, wc_col, sig = _edge_tables(
        wmT_ref[...], embT_ref[...], bm2_ref[...], wqT_ref[...],
        bq2_ref[...], wkT_ref[...], bk2_ref[...])
    ones_row = jnp.ones((1, 128), bf16)
    wcb = wc_col.astype(bf16)
    nbase = [-b.astype(bf16) for b in base]
    sum_b = [jnp.sum(b, keepdims=True) for b in base]
    cb = c_ref[0].astype(bf16)
    CH = 512
    for k in range(LB // CH):
        pk = wcb * cb[:, k * CH:(k + 1) * CH]
        for t in range(4):
            y = jnp.maximum(pk, nbase[t])
            s = jnp.dot(ones_row, y, preferred_element_type=jnp.float32)
            v_ref[pl.ds(t, 1), k * CH:(k + 1) * CH] = sig[t] * (s + sum_b[t])


def _edge_values(c3, wmT, embT, bm2, wqT, bq2, wkT, bk2):
    small = lambda shp: pl.BlockSpec(shp, lambda i: tuple(0 for _ in shp))
    return pl.pallas_call(
        _edge_values_body,
        grid=(E_PAD // LB,),
        in_specs=[
            pl.BlockSpec((1, 1, LB), lambda i: (i, 0, 0)),
            small((128, 11)), small((5, 2)), small((128, 1)),
            small((128, 5)), small((128, 1)), small((128, 5)),
            small((128, 1)),
        ],
        out_specs=pl.BlockSpec((4, LB), lambda i: (0, i)),
        out_shape=jax.ShapeDtypeStruct((4, E_PAD), jnp.float32),
    )(c3, wmT, embT, bm2, wqT, bq2, wkT, bk2)


def _gather_scatter(h_pad, snd1, rcv1, snd3, rcv3, v_flat):
    mesh = plsc.VectorSubcoreMesh(core_axis_name="c", subcore_axis_name="s")

    @functools.partial(
        pl.kernel,
        mesh=mesh,
        compiler_params=pltpu.CompilerParams(needs_layout_passes=False),
        out_type=jax.ShapeDtypeStruct((NC, N_ACC), jnp.float32),
        scratch_types=[
            pltpu.VMEM((N_ACC,), jnp.int32),
            pltpu.VMEM((EPW,), jnp.int32),
            pltpu.VMEM((EPW,), jnp.int32),
            pltpu.VMEM((KCH, 128), jnp.int32),
            pltpu.VMEM((KCH, 128), jnp.int32),
            pltpu.VMEM((4 * EPW,), jnp.float32),
            pltpu.VMEM((EPW,), jnp.float32),
            pltpu.VMEM((EPW,), jnp.float32),
            pltpu.VMEM((N_ACC // NS,), jnp.float32),
            pltpu.VMEM_SHARED((N_ACC,), jnp.float32),
        ],
    )
    def sc_kernel(h_hbm, snd1_hbm, rcv1_hbm, snd3_hbm, rcv3_hbm, v_hbm,
                  out_hbm, htab, s1, r1, s2, r2, vloc, fw1, bw1, zbuf,
                  shared):
        cid = lax.axis_index("c")
        sid = lax.axis_index("s")
        wid = cid * NS + sid
        base_e = wid * EPW
        pltpu.sync_copy(h_hbm, htab)
        pltpu.sync_copy(snd1_hbm.at[pl.ds(base_e, EPW)], s1)
        pltpu.sync_copy(rcv1_hbm.at[pl.ds(base_e, EPW)], r1)
        pltpu.sync_copy(snd3_hbm.at[wid], s2)
        pltpu.sync_copy(rcv3_hbm.at[wid], r2)
        for t in range(4):
            pltpu.sync_copy(v_hbm.at[pl.ds(t * E_PAD + base_e, EPW)],
                            vloc.at[pl.ds(t * EPW, EPW)])

        nz = (N_ACC // NS) // 16

        def zloop(i, carry):
            zbuf[pl.ds(i * 16, 16)] = jnp.zeros((16,), jnp.float32)
            return carry

        lax.fori_loop(0, nz, zloop, 0)
        pltpu.sync_copy(zbuf, shared.at[pl.ds(sid * (N_ACC // NS),
                                              N_ACC // NS)])
        plsc.subcore_barrier()

        iota16 = lax.iota(jnp.int32, 16)

        def cbody(i, carry):
            off = i * 16
            s16 = s1[pl.ds(off, 16)]
            r16 = r1[pl.ds(off, 16)]
            hs = plsc.load_gather(htab, [s16])
            hr = plsc.load_gather(htab, [r16])
            a16 = jnp.right_shift(hs + 1, 1)
            b16 = jnp.right_shift(hr + 1, 1)
            el = off + iota16
            vf = plsc.load_gather(vloc, [(2 * b16 + a16) * EPW + el])
            vb = plsc.load_gather(vloc, [(2 * a16 + b16) * EPW + el])
            fw1[pl.ds(off, 16)] = vf
            bw1[pl.ds(off, 16)] = vb
            return carry

        lax.fori_loop(0, EPW // 16, cbody, 0)

        def sbody(j, carry):
            pltpu.sync_copy(fw1.at[pl.ds(j * 128, 128)],
                            shared.at[r2.at[j]], add=True)
            pltpu.sync_copy(bw1.at[pl.ds(j * 128, 128)],
                            shared.at[s2.at[j]], add=True)
            return carry

        lax.fori_loop(0, KCH, sbody, 0)
        plsc.subcore_barrier()

        @pl.when(sid == 0)
        def _():
            pltpu.sync_copy(shared, out_hbm.at[cid])

    return sc_kernel(h_pad, snd1, rcv1, snd3, rcv3, v_flat)


def _ffn_body(pcol_ref, w_ref, b_ref, o_ref, acc_ref):
    i = pl.program_id(0)
    h_col = pcol_ref[:, 0:1] + pcol_ref[:, 1:2]
    y = lax.dot_general(h_col, w_ref[...],
                        (((0,), (0,)), ((), ())),
                        preferred_element_type=jnp.float32)

    @pl.when(i == 0)
    def _():
        acc_ref[...] = b_ref[...] + y

    @pl.when(i > 0)
    def _():
        acc_ref[...] = acc_ref[...] + y

    @pl.when(i == pl.num_programs(0) - 1)
    def _():
        ya = acc_ref[...]
        scale = 1.0507009873554805
        alpha = 1.6732632423543772
        ysel = scale * jnp.where(ya > 0, ya, alpha * (jnp.exp(ya) - 1.0))
        p = jnp.sum(jnp.exp(ysel))
        o_ref[...] = jnp.broadcast_to(jnp.log(p), (1, 1))


def _ffn(pcol, W_ffn, b2):
    return pl.pallas_call(
        _ffn_body,
        grid=(N // CB,),
        in_specs=[
            pl.BlockSpec((CB, 2), lambda i: (i, 0)),
            pl.BlockSpec((CB, N), lambda i: (i, 0)),
            pl.BlockSpec((1, N), lambda i: (0, 0)),
        ],
        out_specs=pl.BlockSpec((1, 1), lambda i: (0, 0)),
        out_shape=jax.ShapeDtypeStruct((1, 1), jnp.float32),
        scratch_shapes=[pltpu.VMEM((1, N), jnp.float32)],
    )(pcol, W_ffn, b2)


def kernel(h, senders, receivers, couplings, embed, W_mlp, b_mlp, Wq, bq,
           Wk, bk, W_ffn, b_ffn):
    f32 = jnp.float32
    i32 = jnp.int32
    pad = E_PAD - E
    c_pad = jnp.concatenate([couplings.astype(f32), jnp.zeros((pad,), f32)])
    c3 = c_pad.reshape(E_PAD // LB, 1, LB)
    snd1 = jnp.concatenate([senders.astype(i32),
                            jnp.full((pad,), TRASH, i32)])
    rcv1 = jnp.concatenate([receivers.astype(i32),
                            jnp.full((pad,), TRASH, i32)])
    snd3 = snd1.reshape(NW, KCH, 128)
    rcv3 = rcv1.reshape(NW, KCH, 128)
    h_pad = jnp.concatenate([h.astype(i32), jnp.ones((N_ACC - N,), i32)])

    wmT = W_mlp.astype(f32).T
    embT = embed.astype(f32).T
    bm2 = b_mlp.astype(f32)[:, None]
    wqT = Wq.astype(f32).T
    bq2 = bq.astype(f32)[:, None]
    wkT = Wk.astype(f32).T
    bk2 = bk.astype(f32)[:, None]

    v = _edge_values(c3, wmT, embT, bm2, wqT, bq2, wkT, bk2)
    v_flat = v.reshape(4 * E_PAD)
    parts = _gather_scatter(h_pad, snd1, rcv1, snd3, rcv3, v_flat)
    pcol = parts[:, :N].T
    out = _ffn(pcol, W_ffn.astype(f32), b_ffn.astype(f32)[None, :])
    return out[0, 0]


# speedup vs baseline: 1.1043x; 1.1043x over previous
import functools

import jax
import jax.numpy as jnp
from jax import lax
from jax.experimental import pallas as pl
from jax.experimental.pallas import tpu as pltpu
from jax.experimental.pallas import tpu_sc as plsc

N = 10000
E = 320000
F = 128
NC = 2
NS = 16
NW = NC * NS
KCH = 80
EPW = KCH * 128
E_PAD = NW * EPW
LB = 16384
N_ACC = 10240
TRASH = 10100
CB = 400


def _edge_tables(wmT, embT, bm2, wqT, bq2, wkT, bk2):
    f32 = jnp.float32
    e0c = embT[:, 0:1]
    e1c = embT[:, 1:2]
    dcc = e1c - e0c
    WrT = wmT[:, 0:5]
    WsT = wmT[:, 5:10]
    wc_col = wmT[:, 10:11]
    dot = functools.partial(jnp.dot, preferred_element_type=f32)
    Acol = dot(WrT, e0c) + dot(WsT, e0c) + bm2
    Brc = dot(WrT, dcc)
    Bsc = dot(WsT, dcc)
    q0 = dot(wqT, e0c) + bq2
    dq = dot(wqT, dcc)
    k0 = dot(wkT, e0c) + bk2
    dk = dot(wkT, dcc)
    base = [Acol, Acol + Bsc, Acol + Brc, Acol + Brc + Bsc]
    sig = []
    for ir in (0, 1):
        for is_ in (0, 1):
            a = jnp.sum((q0 + is_ * dq) * (k0 + ir * dk))
            sig.append(1.0 / (1.0 + jnp.exp(-a)))
    return base, wc_col, sig


def _edge_values_body(c_ref, wmT_ref, embT_ref, bm2_ref, wqT_ref, bq2_ref,
                      wkT_ref, bk2_ref, v_ref):
    bf16 = jnp.bfloat16
    base, wc_col, sig = _edge_tables(
        wmT_ref[...], embT_ref[...], bm2_ref[...], wqT_ref[...],
        bq2_ref[...], wkT_ref[...], bk2_ref[...])
    ones_row = jnp.ones((1, 128), bf16)
    wcb = wc_col.astype(bf16)
    nbase = [-b.astype(bf16) for b in base]
    sum_b = [jnp.sum(b, keepdims=True) for b in base]
    cb = c_ref[0].astype(bf16)
    CH = 512
    for k in range(LB // CH):
        pk = wcb * cb[:, k * CH:(k + 1) * CH]
        for t in range(4):
            y = jnp.maximum(pk, nbase[t])
            s = jnp.dot(ones_row, y, preferred_element_type=jnp.float32)
            v_ref[pl.ds(t, 1), k * CH:(k + 1) * CH] = sig[t] * (s + sum_b[t])


def _edge_values(c3, wmT, embT, bm2, wqT, bq2, wkT, bk2):
    small = lambda shp: pl.BlockSpec(shp, lambda i: tuple(0 for _ in shp))
    return pl.pallas_call(
        _edge_values_body,
        grid=(E_PAD // LB,),
        in_specs=[
            pl.BlockSpec((1, 1, LB), lambda i: (i, 0, 0)),
            small((128, 11)), small((5, 2)), small((128, 1)),
            small((128, 5)), small((128, 1)), small((128, 5)),
            small((128, 1)),
        ],
        out_specs=pl.BlockSpec((4, LB), lambda i: (0, i)),
        out_shape=jax.ShapeDtypeStruct((4, E_PAD), jnp.float32),
    )(c3, wmT, embT, bm2, wqT, bq2, wkT, bk2)


def _gather_scatter(h_pad, snd3, rcv3, v_flat):
    mesh = plsc.VectorSubcoreMesh(core_axis_name="c", subcore_axis_name="s")

    @functools.partial(
        pl.kernel,
        mesh=mesh,
        compiler_params=pltpu.CompilerParams(needs_layout_passes=False),
        out_type=jax.ShapeDtypeStruct((NC, N_ACC), jnp.float32),
        scratch_types=[
            pltpu.VMEM((N_ACC,), jnp.int32),
            pltpu.VMEM((KCH, 128), jnp.int32),
            pltpu.VMEM((KCH, 128), jnp.int32),
            pltpu.VMEM((4 * EPW,), jnp.float32),
            pltpu.VMEM((EPW,), jnp.float32),
            pltpu.VMEM((EPW,), jnp.float32),
            pltpu.VMEM((N_ACC // NS,), jnp.float32),
            pltpu.VMEM_SHARED((N_ACC,), jnp.float32),
        ],
    )
    def sc_kernel(h_hbm, snd3_hbm, rcv3_hbm, v_hbm,
                  out_hbm, htab, s2, r2, vloc, fw1, bw1, zbuf,
                  shared):
        cid = lax.axis_index("c")
        sid = lax.axis_index("s")
        wid = cid * NS + sid
        base_e = wid * EPW
        pltpu.sync_copy(h_hbm, htab)
        pltpu.sync_copy(snd3_hbm.at[wid], s2)
        pltpu.sync_copy(rcv3_hbm.at[wid], r2)
        for t in range(4):
            pltpu.sync_copy(v_hbm.at[pl.ds(t * E_PAD + base_e, EPW)],
                            vloc.at[pl.ds(t * EPW, EPW)])

        nz = (N_ACC // NS) // 16

        def zloop(i, carry):
            zbuf[pl.ds(i * 16, 16)] = jnp.zeros((16,), jnp.float32)
            return carry

        lax.fori_loop(0, nz, zloop, 0)
        pltpu.sync_copy(zbuf, shared.at[pl.ds(sid * (N_ACC // NS),
                                              N_ACC // NS)])
        plsc.subcore_barrier()

        iota16 = lax.iota(jnp.int32, 16)

        def cbody(i, carry):
            j = lax.shift_right_logical(i, 3)
            off16 = (i & 7) * 16
            off = j * 128 + off16
            s16 = s2[j, pl.ds(off16, 16)]
            r16 = r2[j, pl.ds(off16, 16)]
            hs = plsc.load_gather(htab, [s16])
            hr = plsc.load_gather(htab, [r16])
            a16 = jnp.right_shift(hs + 1, 1)
            b16 = jnp.right_shift(hr + 1, 1)
            el = off + iota16
            vf = plsc.load_gather(vloc, [(2 * b16 + a16) * EPW + el])
            vb = plsc.load_gather(vloc, [(2 * a16 + b16) * EPW + el])
            fw1[pl.ds(off, 16)] = vf
            bw1[pl.ds(off, 16)] = vb
            return carry

        lax.fori_loop(0, EPW // 16, cbody, 0)

        def sbody(j, carry):
            pltpu.sync_copy(fw1.at[pl.ds(j * 128, 128)],
                            shared.at[r2.at[j]], add=True)
            pltpu.sync_copy(bw1.at[pl.ds(j * 128, 128)],
                            shared.at[s2.at[j]], add=True)
            return carry

        lax.fori_loop(0, KCH, sbody, 0)
        plsc.subcore_barrier()

        @pl.when(sid == 0)
        def _():
            pltpu.sync_copy(shared, out_hbm.at[cid])

    return sc_kernel(h_pad, snd3, rcv3, v_flat)


def _ffn_body(pcol_ref, w_ref, b_ref, o_ref, acc_ref):
    i = pl.program_id(0)
    h_col = pcol_ref[:, 0:1] + pcol_ref[:, 1:2]
    y = lax.dot_general(h_col, w_ref[...],
                        (((0,), (0,)), ((), ())),
                        preferred_element_type=jnp.float32)

    @pl.when(i == 0)
    def _():
        acc_ref[...] = b_ref[...] + y

    @pl.when(i > 0)
    def _():
        acc_ref[...] = acc_ref[...] + y

    @pl.when(i == pl.num_programs(0) - 1)
    def _():
        ya = acc_ref[...]
        scale = 1.0507009873554805
        alpha = 1.6732632423543772
        ysel = scale * jnp.where(ya > 0, ya, alpha * (jnp.exp(ya) - 1.0))
        p = jnp.sum(jnp.exp(ysel))
        o_ref[...] = jnp.broadcast_to(jnp.log(p), (1, 1))


def _ffn(pcol, W_ffn, b2):
    return pl.pallas_call(
        _ffn_body,
        grid=(N // CB,),
        in_specs=[
            pl.BlockSpec((CB, 2), lambda i: (i, 0)),
            pl.BlockSpec((CB, N), lambda i: (i, 0)),
            pl.BlockSpec((1, N), lambda i: (0, 0)),
        ],
        out_specs=pl.BlockSpec((1, 1), lambda i: (0, 0)),
        out_shape=jax.ShapeDtypeStruct((1, 1), jnp.float32),
        scratch_shapes=[pltpu.VMEM((1, N), jnp.float32)],
    )(pcol, W_ffn, b2)


def kernel(h, senders, receivers, couplings, embed, W_mlp, b_mlp, Wq, bq,
           Wk, bk, W_ffn, b_ffn):
    f32 = jnp.float32
    i32 = jnp.int32
    pad = E_PAD - E
    c_pad = jnp.concatenate([couplings.astype(f32), jnp.zeros((pad,), f32)])
    c3 = c_pad.reshape(E_PAD // LB, 1, LB)
    snd1 = jnp.concatenate([senders.astype(i32),
                            jnp.full((pad,), TRASH, i32)])
    rcv1 = jnp.concatenate([receivers.astype(i32),
                            jnp.full((pad,), TRASH, i32)])
    snd3 = snd1.reshape(NW, KCH, 128)
    rcv3 = rcv1.reshape(NW, KCH, 128)
    h_pad = jnp.concatenate([h.astype(i32), jnp.ones((N_ACC - N,), i32)])

    wmT = W_mlp.astype(f32).T
    embT = embed.astype(f32).T
    bm2 = b_mlp.astype(f32)[:, None]
    wqT = Wq.astype(f32).T
    bq2 = bq.astype(f32)[:, None]
    wkT = Wk.astype(f32).T
    bk2 = bk.astype(f32)[:, None]

    v = _edge_values(c3, wmT, embT, bm2, wqT, bq2, wkT, bk2)
    v_flat = v.reshape(4 * E_PAD)
    parts = _gather_scatter(h_pad, snd3, rcv3, v_flat)
    pcol = parts[:, :N].T
    out = _ffn(pcol, W_ffn.astype(f32), b_ffn.astype(f32)[None, :])
    return out[0, 0]


# speedup vs baseline: 1.1220x; 1.0160x over previous
import functools

import jax
import jax.numpy as jnp
from jax import lax
from jax.experimental import pallas as pl
from jax.experimental.pallas import tpu as pltpu
from jax.experimental.pallas import tpu_sc as plsc

N = 10000
E = 320000
F = 128
NC = 2
NS = 16
NW = NC * NS
KCH = 80
EPW = KCH * 128
E_PAD = NW * EPW
LB = 16384
N_ACC = 10240
TRASH = 10100
CB = 400


def _edge_tables(wmT, embT, bm2, wqT, bq2, wkT, bk2):
    f32 = jnp.float32
    e0c = embT[:, 0:1]
    e1c = embT[:, 1:2]
    dcc = e1c - e0c
    WrT = wmT[:, 0:5]
    WsT = wmT[:, 5:10]
    wc_col = wmT[:, 10:11]
    dot = functools.partial(jnp.dot, preferred_element_type=f32)
    Acol = dot(WrT, e0c) + dot(WsT, e0c) + bm2
    Brc = dot(WrT, dcc)
    Bsc = dot(WsT, dcc)
    q0 = dot(wqT, e0c) + bq2
    dq = dot(wqT, dcc)
    k0 = dot(wkT, e0c) + bk2
    dk = dot(wkT, dcc)
    base = [Acol, Acol + Bsc, Acol + Brc, Acol + Brc + Bsc]
    sig = []
    for ir in (0, 1):
        for is_ in (0, 1):
            a = jnp.sum((q0 + is_ * dq) * (k0 + ir * dk))
            sig.append(1.0 / (1.0 + jnp.exp(-a)))
    return base, wc_col, sig


def _edge_values_body(c_ref, wmT_ref, embT_ref, bm2_ref, wqT_ref, bq2_ref,
                      wkT_ref, bk2_ref, v_ref):
    bf16 = jnp.bfloat16
    base, wc_col, sig = _edge_tables(
        wmT_ref[...], embT_ref[...], bm2_ref[...], wqT_ref[...],
        bq2_ref[...], wkT_ref[...], bk2_ref[...])
    ones_row = jnp.ones((1, 128), bf16)
    wcb = wc_col.astype(bf16)
    nbase = [-b.astype(bf16) for b in base]
    sum_b = [jnp.sum(b, keepdims=True) for b in base]
    cb = c_ref[0].astype(bf16)
    CH = 512
    for k in range(LB // CH):
        pk = wcb * cb[:, k * CH:(k + 1) * CH]
        for t in range(4):
            y = jnp.maximum(pk, nbase[t])
            s = jnp.dot(ones_row, y, preferred_element_type=jnp.float32)
            v_ref[pl.ds(t, 1), k * CH:(k + 1) * CH] = sig[t] * (s + sum_b[t])


def _edge_values(c3, wmT, embT, bm2, wqT, bq2, wkT, bk2):
    small = lambda shp: pl.BlockSpec(shp, lambda i: tuple(0 for _ in shp))
    return pl.pallas_call(
        _edge_values_body,
        grid=(E_PAD // LB,),
        in_specs=[
            pl.BlockSpec((1, 1, LB), lambda i: (i, 0, 0)),
            small((128, 11)), small((5, 2)), small((128, 1)),
            small((128, 5)), small((128, 1)), small((128, 5)),
            small((128, 1)),
        ],
        out_specs=pl.BlockSpec((4, LB), lambda i: (0, i)),
        out_shape=jax.ShapeDtypeStruct((4, E_PAD), jnp.float32),
    )(c3, wmT, embT, bm2, wqT, bq2, wkT, bk2)


def _gather_scatter(h_pad, snd3, rcv3, v_flat):
    mesh = plsc.VectorSubcoreMesh(core_axis_name="c", subcore_axis_name="s")

    @functools.partial(
        pl.kernel,
        mesh=mesh,
        compiler_params=pltpu.CompilerParams(needs_layout_passes=False),
        out_type=jax.ShapeDtypeStruct((NC, N_ACC), jnp.float32),
        scratch_types=[
            pltpu.VMEM((N_ACC,), jnp.int32),
            pltpu.VMEM((KCH, 128), jnp.int32),
            pltpu.VMEM((KCH, 128), jnp.int32),
            pltpu.VMEM((4 * EPW,), jnp.float32),
            pltpu.VMEM((EPW,), jnp.float32),
            pltpu.VMEM((EPW,), jnp.float32),
            pltpu.VMEM((N_ACC // NS,), jnp.float32),
            pltpu.VMEM_SHARED((N_ACC,), jnp.float32),
            pltpu.SemaphoreType.DMA,
        ],
    )
    def sc_kernel(h_hbm, snd3_hbm, rcv3_hbm, v_hbm,
                  out_hbm, htab, s2, r2, vloc, fw1, bw1, zbuf,
                  shared, dsem):
        cid = lax.axis_index("c")
        sid = lax.axis_index("s")
        wid = cid * NS + sid
        base_e = wid * EPW
        copies = [
            pltpu.async_copy(h_hbm, htab, dsem),
            pltpu.async_copy(snd3_hbm.at[wid], s2, dsem),
            pltpu.async_copy(rcv3_hbm.at[wid], r2, dsem),
        ]
        for t in range(4):
            copies.append(pltpu.async_copy(
                v_hbm.at[pl.ds(t * E_PAD + base_e, EPW)],
                vloc.at[pl.ds(t * EPW, EPW)], dsem))

        nz = (N_ACC // NS) // 16

        def zloop(i, carry):
            zbuf[pl.ds(i * 16, 16)] = jnp.zeros((16,), jnp.float32)
            return carry

        lax.fori_loop(0, nz, zloop, 0)
        pltpu.sync_copy(zbuf, shared.at[pl.ds(sid * (N_ACC // NS),
                                              N_ACC // NS)])
        for c in copies:
            c.wait()
        plsc.subcore_barrier()

        iota16 = lax.iota(jnp.int32, 16)

        def cbody(i, carry):
            j = lax.shift_right_logical(i, 3)
            off16 = (i & 7) * 16
            off = j * 128 + off16
            s16 = s2[j, pl.ds(off16, 16)]
            r16 = r2[j, pl.ds(off16, 16)]
            hs = plsc.load_gather(htab, [s16])
            hr = plsc.load_gather(htab, [r16])
            a16 = jnp.right_shift(hs + 1, 1)
            b16 = jnp.right_shift(hr + 1, 1)
            el = off + iota16
            vf = plsc.load_gather(vloc, [(2 * b16 + a16) * EPW + el])
            vb = plsc.load_gather(vloc, [(2 * a16 + b16) * EPW + el])
            fw1[pl.ds(off, 16)] = vf
            bw1[pl.ds(off, 16)] = vb
            return carry

        lax.fori_loop(0, EPW // 16, cbody, 0, unroll=2)

        def sbody(j, carry):
            df = pltpu.async_copy(fw1.at[pl.ds(j * 128, 128)],
                                  shared.at[r2.at[j]], dsem, add=True)
            db = pltpu.async_copy(bw1.at[pl.ds(j * 128, 128)],
                                  shared.at[s2.at[j]], dsem, add=True)
            df.wait()
            db.wait()
            return carry

        lax.fori_loop(0, KCH, sbody, 0)
        plsc.subcore_barrier()

        @pl.when(sid == 0)
        def _():
            pltpu.sync_copy(shared, out_hbm.at[cid])

    return sc_kernel(h_pad, snd3, rcv3, v_flat)


def _ffn_body(pcol_ref, w_ref, b_ref, o_ref, acc_ref):
    i = pl.program_id(0)
    h_col = pcol_ref[:, 0:1] + pcol_ref[:, 1:2]
    y = lax.dot_general(h_col, w_ref[...],
                        (((0,), (0,)), ((), ())),
                        preferred_element_type=jnp.float32)

    @pl.when(i == 0)
    def _():
        acc_ref[...] = b_ref[...] + y

    @pl.when(i > 0)
    def _():
        acc_ref[...] = acc_ref[...] + y

    @pl.when(i == pl.num_programs(0) - 1)
    def _():
        ya = acc_ref[...]
        scale = 1.0507009873554805
        alpha = 1.6732632423543772
        ysel = scale * jnp.where(ya > 0, ya, alpha * (jnp.exp(ya) - 1.0))
        p = jnp.sum(jnp.exp(ysel))
        o_ref[...] = jnp.broadcast_to(jnp.log(p), (1, 1))


def _ffn(pcol, W_ffn, b2):
    return pl.pallas_call(
        _ffn_body,
        grid=(N // CB,),
        in_specs=[
            pl.BlockSpec((CB, 2), lambda i: (i, 0)),
            pl.BlockSpec((CB, N), lambda i: (i, 0)),
            pl.BlockSpec((1, N), lambda i: (0, 0)),
        ],
        out_specs=pl.BlockSpec((1, 1), lambda i: (0, 0)),
        out_shape=jax.ShapeDtypeStruct((1, 1), jnp.float32),
        scratch_shapes=[pltpu.VMEM((1, N), jnp.float32)],
    )(pcol, W_ffn, b2)


def kernel(h, senders, receivers, couplings, embed, W_mlp, b_mlp, Wq, bq,
           Wk, bk, W_ffn, b_ffn):
    f32 = jnp.float32
    i32 = jnp.int32
    pad = E_PAD - E
    c_pad = jnp.concatenate([couplings.astype(f32), jnp.zeros((pad,), f32)])
    c3 = c_pad.reshape(E_PAD // LB, 1, LB)
    snd1 = jnp.concatenate([senders.astype(i32),
                            jnp.full((pad,), TRASH, i32)])
    rcv1 = jnp.concatenate([receivers.astype(i32),
                            jnp.full((pad,), TRASH, i32)])
    snd3 = snd1.reshape(NW, KCH, 128)
    rcv3 = rcv1.reshape(NW, KCH, 128)
    h_pad = jnp.concatenate([h.astype(i32), jnp.ones((N_ACC - N,), i32)])

    wmT = W_mlp.astype(f32).T
    embT = embed.astype(f32).T
    bm2 = b_mlp.astype(f32)[:, None]
    wqT = Wq.astype(f32).T
    bq2 = bq.astype(f32)[:, None]
    wkT = Wk.astype(f32).T
    bk2 = bk.astype(f32)[:, None]

    v = _edge_values(c3, wmT, embT, bm2, wqT, bq2, wkT, bk2)
    v_flat = v.reshape(4 * E_PAD)
    parts = _gather_scatter(h_pad, snd3, rcv3, v_flat)
    pcol = parts[:, :N].T
    out = _ffn(pcol, W_ffn.astype(f32), b_ffn.astype(f32)[None, :])
    return out[0, 0]


# speedup vs baseline: 1.1840x; 1.0552x over previous
import functools

import jax
import jax.numpy as jnp
from jax import lax
from jax.experimental import pallas as pl
from jax.experimental.pallas import tpu as pltpu
from jax.experimental.pallas import tpu_sc as plsc

N = 10000
E = 320000
F = 128
NC = 2
NS = 16
NW = NC * NS
KCH = 80
EPW = KCH * 128
E_PAD = NW * EPW
LB = 16384
N_ACC = 10240
TRASH = 10100
CB = 400


def _edge_tables(wmT, embT, bm2, wqT, bq2, wkT, bk2):
    f32 = jnp.float32
    e0c = embT[:, 0:1]
    e1c = embT[:, 1:2]
    dcc = e1c - e0c
    WrT = wmT[:, 0:5]
    WsT = wmT[:, 5:10]
    wc_col = wmT[:, 10:11]
    dot = functools.partial(jnp.dot, preferred_element_type=f32)
    Acol = dot(WrT, e0c) + dot(WsT, e0c) + bm2
    Brc = dot(WrT, dcc)
    Bsc = dot(WsT, dcc)
    q0 = dot(wqT, e0c) + bq2
    dq = dot(wqT, dcc)
    k0 = dot(wkT, e0c) + bk2
    dk = dot(wkT, dcc)
    base = [Acol, Acol + Bsc, Acol + Brc, Acol + Brc + Bsc]
    sig = []
    for ir in (0, 1):
        for is_ in (0, 1):
            a = jnp.sum((q0 + is_ * dq) * (k0 + ir * dk))
            sig.append(1.0 / (1.0 + jnp.exp(-a)))
    return base, wc_col, sig


def _edge_values_body(c_ref, wmT_ref, embT_ref, bm2_ref, wqT_ref, bq2_ref,
                      wkT_ref, bk2_ref, v_ref):
    bf16 = jnp.bfloat16
    base, wc_col, sig = _edge_tables(
        wmT_ref[...], embT_ref[...], bm2_ref[...], wqT_ref[...],
        bq2_ref[...], wkT_ref[...], bk2_ref[...])
    ones_row = jnp.ones((1, 128), bf16)
    wcb = wc_col.astype(bf16)
    nbase = [-b.astype(bf16) for b in base]
    sum_b = [jnp.sum(b, keepdims=True) for b in base]
    cb = c_ref[0].astype(bf16)
    CH = 512
    for k in range(LB // CH):
        pk = wcb * cb[:, k * CH:(k + 1) * CH]
        for t in range(4):
            y = jnp.maximum(pk, nbase[t])
            s = jnp.dot(ones_row, y, preferred_element_type=jnp.float32)
            v_ref[pl.ds(t, 1), k * CH:(k + 1) * CH] = sig[t] * (s + sum_b[t])


def _edge_values(c3, wmT, embT, bm2, wqT, bq2, wkT, bk2):
    n_blk = c3.shape[0]
    small = lambda shp: pl.BlockSpec(shp, lambda i: tuple(0 for _ in shp))
    return pl.pallas_call(
        _edge_values_body,
        grid=(n_blk,),
        in_specs=[
            pl.BlockSpec((1, 1, LB), lambda i: (i, 0, 0)),
            small((128, 11)), small((5, 2)), small((128, 1)),
            small((128, 5)), small((128, 1)), small((128, 5)),
            small((128, 1)),
        ],
        out_specs=pl.BlockSpec((4, LB), lambda i: (0, i)),
        out_shape=jax.ShapeDtypeStruct((4, n_blk * LB), jnp.float32),
    )(c3, wmT, embT, bm2, wqT, bq2, wkT, bk2)


def _gather_scatter(h_pad, snd3, rcv3, v_flat, kch):
    epw = kch * 128
    e_pad = NW * epw
    mesh = plsc.VectorSubcoreMesh(core_axis_name="c", subcore_axis_name="s")

    @functools.partial(
        pl.kernel,
        mesh=mesh,
        compiler_params=pltpu.CompilerParams(needs_layout_passes=False),
        out_type=jax.ShapeDtypeStruct((NC, N_ACC), jnp.float32),
        scratch_types=[
            pltpu.VMEM((N_ACC,), jnp.int32),
            pltpu.VMEM((kch, 128), jnp.int32),
            pltpu.VMEM((kch, 128), jnp.int32),
            pltpu.VMEM((4 * epw,), jnp.float32),
            pltpu.VMEM((epw,), jnp.float32),
            pltpu.VMEM((epw,), jnp.float32),
            pltpu.VMEM((N_ACC // NS,), jnp.float32),
            pltpu.VMEM_SHARED((N_ACC,), jnp.float32),
            pltpu.SemaphoreType.DMA,
        ],
    )
    def sc_kernel(h_hbm, snd3_hbm, rcv3_hbm, v_hbm,
                  out_hbm, htab, s2, r2, vloc, fw1, bw1, zbuf,
                  shared, dsem):
        cid = lax.axis_index("c")
        sid = lax.axis_index("s")
        wid = cid * NS + sid
        base_e = wid * epw
        copies = [
            pltpu.async_copy(h_hbm, htab, dsem),
            pltpu.async_copy(snd3_hbm.at[wid], s2, dsem),
            pltpu.async_copy(rcv3_hbm.at[wid], r2, dsem),
        ]
        for t in range(4):
            copies.append(pltpu.async_copy(
                v_hbm.at[pl.ds(t * e_pad + base_e, epw)],
                vloc.at[pl.ds(t * epw, epw)], dsem))

        nz = (N_ACC // NS) // 16

        def zloop(i, carry):
            zbuf[pl.ds(i * 16, 16)] = jnp.zeros((16,), jnp.float32)
            return carry

        lax.fori_loop(0, nz, zloop, 0)
        pltpu.sync_copy(zbuf, shared.at[pl.ds(sid * (N_ACC // NS),
                                              N_ACC // NS)])
        for c in copies:
            c.wait()
        plsc.subcore_barrier()

        iota16 = lax.iota(jnp.int32, 16)

        def cbody(i, carry):
            j = lax.shift_right_logical(i, 3)
            off16 = (i & 7) * 16
            off = j * 128 + off16
            s16 = s2[j, pl.ds(off16, 16)]
            r16 = r2[j, pl.ds(off16, 16)]
            hs = plsc.load_gather(htab, [s16])
            hr = plsc.load_gather(htab, [r16])
            a16 = jnp.right_shift(hs + 1, 1)
            b16 = jnp.right_shift(hr + 1, 1)
            el = off + iota16
            vf = plsc.load_gather(vloc, [(2 * b16 + a16) * epw + el])
            vb = plsc.load_gather(vloc, [(2 * a16 + b16) * epw + el])
            fw1[pl.ds(off, 16)] = vf
            bw1[pl.ds(off, 16)] = vb
            return carry

        lax.fori_loop(0, epw // 16, cbody, 0, unroll=2)

        def sbody(j, carry):
            df = pltpu.async_copy(fw1.at[pl.ds(j * 128, 128)],
                                  shared.at[r2.at[j]], dsem, add=True)
            db = pltpu.async_copy(bw1.at[pl.ds(j * 128, 128)],
                                  shared.at[s2.at[j]], dsem, add=True)
            df.wait()
            db.wait()
            return carry

        lax.fori_loop(0, kch, sbody, 0)
        plsc.subcore_barrier()

        @pl.when(sid == 0)
        def _():
            pltpu.sync_copy(shared, out_hbm.at[cid])

    return sc_kernel(h_pad, snd3, rcv3, v_flat)


def _ffn_body(pcol_ref, w_ref, b_ref, o_ref, acc_ref):
    i = pl.program_id(0)
    h_col = ((pcol_ref[:, 0:1] + pcol_ref[:, 1:2])
             + (pcol_ref[:, 2:3] + pcol_ref[:, 3:4]))
    y = lax.dot_general(h_col, w_ref[...],
                        (((0,), (0,)), ((), ())),
                        preferred_element_type=jnp.float32)

    @pl.when(i == 0)
    def _():
        acc_ref[...] = b_ref[...] + y

    @pl.when(i > 0)
    def _():
        acc_ref[...] = acc_ref[...] + y

    @pl.when(i == pl.num_programs(0) - 1)
    def _():
        ya = acc_ref[...]
        scale = 1.0507009873554805
        alpha = 1.6732632423543772
        ysel = scale * jnp.where(ya > 0, ya, alpha * (jnp.exp(ya) - 1.0))
        p = jnp.sum(jnp.exp(ysel))
        o_ref[...] = jnp.broadcast_to(jnp.log(p), (1, 1))


def _ffn(pcol, W_ffn, b2):
    return pl.pallas_call(
        _ffn_body,
        grid=(N // CB,),
        in_specs=[
            pl.BlockSpec((CB, 4), lambda i: (i, 0)),
            pl.BlockSpec((CB, N), lambda i: (i, 0)),
            pl.BlockSpec((1, N), lambda i: (0, 0)),
        ],
        out_specs=pl.BlockSpec((1, 1), lambda i: (0, 0)),
        out_shape=jax.ShapeDtypeStruct((1, 1), jnp.float32),
        scratch_shapes=[pltpu.VMEM((1, N), jnp.float32)],
    )(pcol, W_ffn, b2)


def kernel(h, senders, receivers, couplings, embed, W_mlp, b_mlp, Wq, bq,
           Wk, bk, W_ffn, b_ffn):
    f32 = jnp.float32
    i32 = jnp.int32
    pad = E_PAD - E
    c_pad = jnp.concatenate([couplings.astype(f32), jnp.zeros((pad,), f32)])
    c4 = c_pad.reshape(2, E_PAD // (2 * LB), 1, LB)
    snd1 = jnp.concatenate([senders.astype(i32),
                            jnp.full((pad,), TRASH, i32)])
    rcv1 = jnp.concatenate([receivers.astype(i32),
                            jnp.full((pad,), TRASH, i32)])
    snd4 = snd1.reshape(2, NW, KCH // 2, 128)
    rcv4 = rcv1.reshape(2, NW, KCH // 2, 128)
    h_pad = jnp.concatenate([h.astype(i32), jnp.ones((N_ACC - N,), i32)])

    wmT = W_mlp.astype(f32).T
    embT = embed.astype(f32).T
    bm2 = b_mlp.astype(f32)[:, None]
    wqT = Wq.astype(f32).T
    bq2 = bq.astype(f32)[:, None]
    wkT = Wk.astype(f32).T
    bk2 = bk.astype(f32)[:, None]

    va = _edge_values(c4[0], wmT, embT, bm2, wqT, bq2, wkT, bk2)
    vb = _edge_values(c4[1], wmT, embT, bm2, wqT, bq2, wkT, bk2)
    p1 = _gather_scatter(h_pad, snd4[0], rcv4[0],
                         va.reshape(4 * E_PAD // 2), KCH // 2)
    p2 = _gather_scatter(h_pad, snd4[1], rcv4[1],
                         vb.reshape(4 * E_PAD // 2), KCH // 2)
    pcol = jnp.concatenate([p1, p2])[:, :N].T
    out = _ffn(pcol, W_ffn.astype(f32), b_ffn.astype(f32)[None, :])
    return out[0, 0]


# speedup vs baseline: 1.2094x; 1.0214x over previous
import functools

import jax
import jax.numpy as jnp
from jax import lax
from jax.experimental import pallas as pl
from jax.experimental.pallas import tpu as pltpu
from jax.experimental.pallas import tpu_sc as plsc

N = 10000
E = 320000
F = 128
NC = 2
NS = 16
NW = NC * NS
KCH = 80
EPW = KCH * 128
E_PAD = NW * EPW
LB = 32768
N_ACC = 10240
TRASH = 10100
CB = 400


def _edge_tables(wmT, embT, bm2, wqT, bq2, wkT, bk2):
    f32 = jnp.float32
    e0c = embT[:, 0:1]
    e1c = embT[:, 1:2]
    dcc = e1c - e0c
    WrT = wmT[:, 0:5]
    WsT = wmT[:, 5:10]
    wc_col = wmT[:, 10:11]
    dot = functools.partial(jnp.dot, preferred_element_type=f32)
    Acol = dot(WrT, e0c) + dot(WsT, e0c) + bm2
    Brc = dot(WrT, dcc)
    Bsc = dot(WsT, dcc)
    q0 = dot(wqT, e0c) + bq2
    dq = dot(wqT, dcc)
    k0 = dot(wkT, e0c) + bk2
    dk = dot(wkT, dcc)
    base = [Acol, Acol + Bsc, Acol + Brc, Acol + Brc + Bsc]
    sig = []
    for ir in (0, 1):
        for is_ in (0, 1):
            a = jnp.sum((q0 + is_ * dq) * (k0 + ir * dk))
            sig.append(1.0 / (1.0 + jnp.exp(-a)))
    return base, wc_col, sig


def _edge_values_body(c_ref, wmT_ref, embT_ref, bm2_ref, wqT_ref, bq2_ref,
                      wkT_ref, bk2_ref, v_ref):
    bf16 = jnp.bfloat16
    base, wc_col, sig = _edge_tables(
        wmT_ref[...], embT_ref[...], bm2_ref[...], wqT_ref[...],
        bq2_ref[...], wkT_ref[...], bk2_ref[...])
    ones_row = jnp.ones((1, 128), bf16)
    wcb = wc_col.astype(bf16)
    nbase = [-b.astype(bf16) for b in base]
    sum_b = [jnp.sum(b, keepdims=True) for b in base]
    cb = c_ref[0].astype(bf16)
    CH = 512
    for k in range(LB // CH):
        pk = wcb * cb[:, k * CH:(k + 1) * CH]
        for t in range(4):
            y = jnp.maximum(pk, nbase[t])
            s = jnp.dot(ones_row, y, preferred_element_type=jnp.float32)
            v_ref[pl.ds(t, 1), k * CH:(k + 1) * CH] = sig[t] * (s + sum_b[t])


def _edge_values(c3, wmT, embT, bm2, wqT, bq2, wkT, bk2):
    n_blk = c3.shape[0]
    small = lambda shp: pl.BlockSpec(shp, lambda i: tuple(0 for _ in shp))
    return pl.pallas_call(
        _edge_values_body,
        grid=(n_blk,),
        in_specs=[
            pl.BlockSpec((1, 1, LB), lambda i: (i, 0, 0)),
            small((128, 11)), small((5, 2)), small((128, 1)),
            small((128, 5)), small((128, 1)), small((128, 5)),
            small((128, 1)),
        ],
        out_specs=pl.BlockSpec((4, LB), lambda i: (0, i)),
        out_shape=jax.ShapeDtypeStruct((4, n_blk * LB), jnp.float32),
    )(c3, wmT, embT, bm2, wqT, bq2, wkT, bk2)


def _gather_scatter(h_pad, snd3, rcv3, v_flat, kch):
    epw = kch * 128
    e_pad = NW * epw
    mesh = plsc.VectorSubcoreMesh(core_axis_name="c", subcore_axis_name="s")

    @functools.partial(
        pl.kernel,
        mesh=mesh,
        compiler_params=pltpu.CompilerParams(needs_layout_passes=False),
        out_type=jax.ShapeDtypeStruct((NC, N_ACC), jnp.float32),
        scratch_types=[
            pltpu.VMEM((N_ACC,), jnp.int32),
            pltpu.VMEM((kch, 128), jnp.int32),
            pltpu.VMEM((kch, 128), jnp.int32),
            pltpu.VMEM((4 * epw,), jnp.float32),
            pltpu.VMEM((epw,), jnp.float32),
            pltpu.VMEM((epw,), jnp.float32),
            pltpu.VMEM((N_ACC // NS,), jnp.float32),
            pltpu.VMEM_SHARED((N_ACC,), jnp.float32),
            pltpu.SemaphoreType.DMA,
        ],
    )
    def sc_kernel(h_hbm, snd3_hbm, rcv3_hbm, v_hbm,
                  out_hbm, htab, s2, r2, vloc, fw1, bw1, zbuf,
                  shared, dsem):
        cid = lax.axis_index("c")
        sid = lax.axis_index("s")
        wid = cid * NS + sid
        base_e = wid * epw
        copies = [
            pltpu.async_copy(h_hbm, htab, dsem),
            pltpu.async_copy(snd3_hbm.at[wid], s2, dsem),
            pltpu.async_copy(rcv3_hbm.at[wid], r2, dsem),
        ]
        for t in range(4):
            copies.append(pltpu.async_copy(
                v_hbm.at[pl.ds(t * e_pad + base_e, epw)],
                vloc.at[pl.ds(t * epw, epw)], dsem))

        nz = (N_ACC // NS) // 16

        def zloop(i, carry):
            zbuf[pl.ds(i * 16, 16)] = jnp.zeros((16,), jnp.float32)
            return carry

        lax.fori_loop(0, nz, zloop, 0)
        pltpu.sync_copy(zbuf, shared.at[pl.ds(sid * (N_ACC // NS),
                                              N_ACC // NS)])
        for c in copies:
            c.wait()
        plsc.subcore_barrier()

        iota16 = lax.iota(jnp.int32, 16)

        def cbody(i, carry):
            j = lax.shift_right_logical(i, 3)
            off16 = (i & 7) * 16
            off = j * 128 + off16
            s16 = s2[j, pl.ds(off16, 16)]
            r16 = r2[j, pl.ds(off16, 16)]
            hs = plsc.load_gather(htab, [s16])
            hr = plsc.load_gather(htab, [r16])
            a16 = jnp.right_shift(hs + 1, 1)
            b16 = jnp.right_shift(hr + 1, 1)
            el = off + iota16
            vf = plsc.load_gather(vloc, [(2 * b16 + a16) * epw + el])
            vb = plsc.load_gather(vloc, [(2 * a16 + b16) * epw + el])
            fw1[pl.ds(off, 16)] = vf
            bw1[pl.ds(off, 16)] = vb
            return carry

        lax.fori_loop(0, epw // 16, cbody, 0, unroll=2)

        def sbody(j, carry):
            df = pltpu.async_copy(fw1.at[pl.ds(j * 128, 128)],
                                  shared.at[r2.at[j]], dsem, add=True)
            db = pltpu.async_copy(bw1.at[pl.ds(j * 128, 128)],
                                  shared.at[s2.at[j]], dsem, add=True)
            df.wait()
            db.wait()
            return carry

        lax.fori_loop(0, kch, sbody, 0)
        plsc.subcore_barrier()

        @pl.when(sid == 0)
        def _():
            pltpu.sync_copy(shared, out_hbm.at[cid])

    return sc_kernel(h_pad, snd3, rcv3, v_flat)


def _ffn_body(pcol_ref, w_ref, b_ref, o_ref, acc_ref):
    i = pl.program_id(0)
    h_col = ((pcol_ref[:, 0:1] + pcol_ref[:, 1:2])
             + (pcol_ref[:, 2:3] + pcol_ref[:, 3:4]))
    y = lax.dot_general(h_col, w_ref[...],
                        (((0,), (0,)), ((), ())),
                        preferred_element_type=jnp.float32)

    @pl.when(i == 0)
    def _():
        acc_ref[...] = b_ref[...] + y

    @pl.when(i > 0)
    def _():
        acc_ref[...] = acc_ref[...] + y

    @pl.when(i == pl.num_programs(0) - 1)
    def _():
        ya = acc_ref[...]
        scale = 1.0507009873554805
        alpha = 1.6732632423543772
        ysel = scale * jnp.where(ya > 0, ya, alpha * (jnp.exp(ya) - 1.0))
        p = jnp.sum(jnp.exp(ysel))
        o_ref[...] = jnp.broadcast_to(jnp.log(p), (1, 1))


def _ffn(pcol, W_ffn, b2):
    return pl.pallas_call(
        _ffn_body,
        grid=(N // CB,),
        in_specs=[
            pl.BlockSpec((CB, 4), lambda i: (i, 0)),
            pl.BlockSpec((CB, N), lambda i: (i, 0)),
            pl.BlockSpec((1, N), lambda i: (0, 0)),
        ],
        out_specs=pl.BlockSpec((1, 1), lambda i: (0, 0)),
        out_shape=jax.ShapeDtypeStruct((1, 1), jnp.float32),
        scratch_shapes=[pltpu.VMEM((1, N), jnp.float32)],
    )(pcol, W_ffn, b2)


def kernel(h, senders, receivers, couplings, embed, W_mlp, b_mlp, Wq, bq,
           Wk, bk, W_ffn, b_ffn):
    f32 = jnp.float32
    i32 = jnp.int32
    pad = E_PAD - E
    c_pad = jnp.concatenate([couplings.astype(f32), jnp.zeros((pad,), f32)])
    c4 = c_pad.reshape(2, E_PAD // (2 * LB), 1, LB)
    snd1 = jnp.concatenate([senders.astype(i32),
                            jnp.full((pad,), TRASH, i32)])
    rcv1 = jnp.concatenate([receivers.astype(i32),
                            jnp.full((pad,), TRASH, i32)])
    snd4 = snd1.reshape(2, NW, KCH // 2, 128)
    rcv4 = rcv1.reshape(2, NW, KCH // 2, 128)
    h_pad = jnp.concatenate([h.astype(i32), jnp.ones((N_ACC - N,), i32)])

    wmT = W_mlp.astype(f32).T
    embT = embed.astype(f32).T
    bm2 = b_mlp.astype(f32)[:, None]
    wqT = Wq.astype(f32).T
    bq2 = bq.astype(f32)[:, None]
    wkT = Wk.astype(f32).T
    bk2 = bk.astype(f32)[:, None]

    va = _edge_values(c4[0], wmT, embT, bm2, wqT, bq2, wkT, bk2)
    vb = _edge_values(c4[1], wmT, embT, bm2, wqT, bq2, wkT, bk2)
    p1 = _gather_scatter(h_pad, snd4[0], rcv4[0],
                         va.reshape(4 * E_PAD // 2), KCH // 2)
    p2 = _gather_scatter(h_pad, snd4[1], rcv4[1],
                         vb.reshape(4 * E_PAD // 2), KCH // 2)
    pcol = jnp.concatenate([p1, p2])[:, :N].T
    out = _ffn(pcol, W_ffn.astype(f32), b_ffn.astype(f32)[None, :])
    return out[0, 0]
